# baseline (speedup 1.0000x reference)
import functools

import jax
import jax.numpy as jnp
from jax import lax
from jax.experimental import pallas as pl
from jax.experimental.pallas import tpu as pltpu
from jax.experimental.pallas import tpu_sc as plsc

T_W = 128
WIN = 128
NW = 32
EB = 2048
G = 64
NEG = -3e38


def _table_kernel(x, pos, batch, W, b):
    N, D_IN = x.shape
    DIM = W.shape[0]
    BLK = 1000
    grid = N // BLK

    def body(x_ref, pos_ref, bat_ref, w_ref, b_ref, t_ref):
        h = jnp.dot(x_ref[...], w_ref[...].T,
                    preferred_element_type=jnp.float32) + b_ref[...]
        hb = lax.bitcast_convert_type(h, jnp.int32) + 0x8000
        hi = jnp.bitwise_and(hb[:, :DIM // 2], jnp.int32(-65536))
        lo = jnp.bitwise_and(jnp.right_shift(hb[:, DIM // 2:], 16),
                             jnp.int32(0xFFFF))
        packed = jnp.bitwise_or(hi, lo)
        posb = lax.bitcast_convert_type(pos_ref[...], jnp.int32)
        z = jnp.zeros((BLK, T_W - DIM // 2 - 4), jnp.int32)
        t_ref[...] = jnp.concatenate([packed, posb, bat_ref[...], z], axis=1)

    return pl.pallas_call(
        body,
        grid=(grid,),
        in_specs=[
            pl.BlockSpec((BLK, D_IN), lambda i: (i, 0)),
            pl.BlockSpec((BLK, 3), lambda i: (i, 0)),
            pl.BlockSpec((BLK, 1), lambda i: (i, 0)),
            pl.BlockSpec((DIM, D_IN), lambda i: (0, 0)),
            pl.BlockSpec((1, DIM), lambda i: (0, 0)),
        ],
        out_specs=pl.BlockSpec((BLK, T_W), lambda i: (i, 0)),
        out_shape=jax.ShapeDtypeStruct((N, T_W), jnp.int32),
    )(x, pos, batch[:, None], W, b[None, :])


def _sc_gather(table, idx_all):
    M = idx_all.shape[0]
    mesh = plsc.VectorSubcoreMesh(core_axis_name="c", subcore_axis_name="s")
    per_w = M // WIN // NW

    @functools.partial(
        pl.kernel,
        out_type=jax.ShapeDtypeStruct((M, T_W), jnp.int32),
        mesh=mesh,
    )
    def k(t_hbm, i_hbm, o_hbm):
        def body(i_vmem, o_vmem):
            pltpu.sync_copy(t_hbm.at[i_vmem.at[0]], o_vmem)

        pltpu.emit_pipeline(
            body,
            grid=(NW, per_w),
            in_specs=[pl.BlockSpec((1, WIN), lambda w, i: (0, w * per_w + i))],
            out_specs=[pl.BlockSpec((WIN, T_W), lambda w, i: (w * per_w + i, 0))],
            core_axis_name=("c", "s"),
            dimension_semantics=(pltpu.PARALLEL, pltpu.ARBITRARY),
        )(i_hbm, o_hbm)

    return k(table, idx_all.reshape(1, M))


def _edge_kernel(tstd, w_rbf_t, attn, w_read_t, b_read, e_real, n_blocks):
    E_pad = tstd.shape[0] // 2
    DIM = attn.shape[1]
    NG = w_rbf_t.shape[0]
    offs = [5.0 * k / (NG - 1) for k in range(NG)]
    gap = offs[1] - offs[0]
    coeff = -0.5 / (gap * gap)

    def body(ts_ref, td_ref, wr_ref, at_ref, wread_ref, bread_ref,
             sum_ref, atom_ref, be_ref):
        i = pl.program_id(0)
        ts = ts_ref[...]
        td = td_ref[...]
        H2 = DIM // 2

        def unpack_h(t):
            w = t[:, :H2]
            h1 = lax.bitcast_convert_type(
                jnp.bitwise_and(w, jnp.int32(-65536)), jnp.float32)
            h2 = lax.bitcast_convert_type(
                jnp.left_shift(w, 16), jnp.float32)
            return jnp.concatenate([h1, h2], axis=1)

        hs = unpack_h(ts) + unpack_h(td)
        e1 = jnp.where(hs > 0, hs, jnp.exp(hs) - 1.0)
        d2 = jnp.zeros((EB, 1), jnp.float32)
        for c in range(3):
            dv = lax.bitcast_convert_type(
                ts[:, H2 + c:H2 + c + 1], jnp.float32) - \
                lax.bitcast_convert_type(
                td[:, H2 + c:H2 + c + 1], jnp.float32)
            d2 = d2 + dv * dv
        dist = jnp.maximum(jnp.sqrt(d2), 0.1)
        emd = jnp.zeros((EB, DIM), jnp.float32)
        for k in range(NG):
            rk = jnp.exp(coeff * (dist - offs[k]) ** 2)
            emd = emd + rk * wr_ref[k:k + 1, :]
        e2 = e1 * emd
        z = e2 * at_ref[...]
        atom = jnp.where(z > 0, z, jnp.exp(z) - 1.0)
        logit = jnp.sum(atom * wread_ref[...], axis=1, keepdims=True)
        score = jax.nn.sigmoid(logit + bread_ref[...])
        rows = i * EB + lax.broadcasted_iota(jnp.int32, (EB, 1), 0)
        valid = rows < e_real
        y = jnp.where(valid, atom * score, 0.0)
        bi = ts[:, H2 + 3:H2 + 4]
        giota = lax.broadcasted_iota(jnp.int32, (1, G), 1)
        onehot = (bi == giota)
        part = lax.dot_general(onehot.astype(jnp.float32), y,
                               (((0,), (0,)), ((), ())),
                               preferred_element_type=jnp.float32)

        @pl.when(i == 0)
        def _():
            sum_ref[...] = jnp.zeros_like(sum_ref)

        sum_ref[...] += part
        atom_ref[...] = jnp.where(valid, atom, NEG).astype(jnp.bfloat16)
        be_ref[...] = bi.reshape(1, EB)

    return pl.pallas_call(
        body,
        grid=(n_blocks,),
        in_specs=[
            pl.BlockSpec((EB, T_W), lambda i: (i, 0)),
            pl.BlockSpec((EB, T_W), lambda i, nb=n_blocks: (nb + i, 0)),
            pl.BlockSpec((NG, DIM), lambda i: (0, 0)),
            pl.BlockSpec((1, DIM), lambda i: (0, 0)),
            pl.BlockSpec((1, DIM), lambda i: (0, 0)),
            pl.BlockSpec((1, 1), lambda i: (0, 0)),
        ],
        out_specs=[
            pl.BlockSpec((G, DIM), lambda i: (0, 0)),
            pl.BlockSpec((EB, DIM), lambda i: (i, 0)),
            pl.BlockSpec((1, EB), lambda i: (0, i)),
        ],
        out_shape=[
            jax.ShapeDtypeStruct((G, DIM), jnp.float32),
            jax.ShapeDtypeStruct((E_pad, DIM), jnp.bfloat16),
            jax.ShapeDtypeStruct((1, E_pad), jnp.int32),
        ],
        compiler_params=pltpu.CompilerParams(
            dimension_semantics=("arbitrary",)),
    )(tstd, tstd, w_rbf_t, attn, w_read_t, b_read)


def kernel(x, pos, edges_0, batch, W, b, attn, W_rbf, w_read, b_read):
    N, D_IN = x.shape
    DIM = W.shape[0]
    E = edges_0.shape[0]

    src = edges_0[:, 0]
    dst = edges_0[:, 1]
    NCHUNK = 4
    unit = WIN * NW * NCHUNK
    E_pad = ((E + unit - 1) // unit) * unit
    pad = E_pad - E
    zpad = jnp.zeros((pad,), jnp.int32)
    src_p = jnp.concatenate([src, zpad])
    dst_p = jnp.concatenate([dst, zpad])
    CE = E_pad // NCHUNK

    table = _table_kernel(x, pos, batch, W, b)

    sums, maxs = [], []
    for c in range(NCHUNK):
        sl = slice(c * CE, (c + 1) * CE)
        idx_c = jnp.concatenate([src_p[sl], dst_p[sl]])
        tstd_c = _sc_gather(table, idx_c)
        e_real_c = max(0, min(E - c * CE, CE))
        sum_c, atom_c, be_c = _edge_kernel(
            tstd_c, W_rbf.T, attn, w_read.T, b_read[None, :],
            e_real_c, CE // EB)
        sums.append(sum_c)
        maxs.append(jax.ops.segment_max(
            atom_c, be_c.reshape(-1), num_segments=G))

    out_sum = sums[0] + sums[1] + sums[2] + sums[3]
    out_max = jnp.maximum(jnp.maximum(maxs[0], maxs[1]),
                          jnp.maximum(maxs[2], maxs[3])).astype(jnp.float32)
    return jnp.concatenate([out_sum, out_max], axis=1)


# speedup vs baseline: 1.1708x; 1.1708x over previous
import functools

import jax
import jax.numpy as jnp
from jax import lax
from jax.experimental import pallas as pl
from jax.experimental.pallas import tpu as pltpu
from jax.experimental.pallas import tpu_sc as plsc

T_W = 128
WIN = 128
NW = 32
EB = 2048
G = 64
NEG = -3e38


def _table_kernel(x, pos, batch, W, b):
    N, D_IN = x.shape
    DIM = W.shape[0]
    BLK = 1000
    grid = N // BLK

    def body(x_ref, pos_ref, bat_ref, w_ref, b_ref, t_ref):
        h = jnp.dot(x_ref[...], w_ref[...].T,
                    preferred_element_type=jnp.float32) + b_ref[...]
        hb = lax.bitcast_convert_type(h, jnp.int32) + 0x8000
        hi = jnp.bitwise_and(hb[:, :DIM // 2], jnp.int32(-65536))
        lo = jnp.bitwise_and(jnp.right_shift(hb[:, DIM // 2:], 16),
                             jnp.int32(0xFFFF))
        packed = jnp.bitwise_or(hi, lo)
        posb = lax.bitcast_convert_type(pos_ref[...], jnp.int32)
        z = jnp.zeros((BLK, T_W - DIM // 2 - 4), jnp.int32)
        t_ref[...] = jnp.concatenate([packed, posb, bat_ref[...], z], axis=1)

    return pl.pallas_call(
        body,
        grid=(grid,),
        in_specs=[
            pl.BlockSpec((BLK, D_IN), lambda i: (i, 0)),
            pl.BlockSpec((BLK, 3), lambda i: (i, 0)),
            pl.BlockSpec((BLK, 1), lambda i: (i, 0)),
            pl.BlockSpec((DIM, D_IN), lambda i: (0, 0)),
            pl.BlockSpec((1, DIM), lambda i: (0, 0)),
        ],
        out_specs=pl.BlockSpec((BLK, T_W), lambda i: (i, 0)),
        out_shape=jax.ShapeDtypeStruct((N, T_W), jnp.int32),
    )(x, pos, batch[:, None], W, b[None, :])


def _sc_gather(table, idx_all):
    M = idx_all.shape[0]
    mesh = plsc.VectorSubcoreMesh(core_axis_name="c", subcore_axis_name="s")
    per_w = M // WIN // NW

    @functools.partial(
        pl.kernel,
        out_type=jax.ShapeDtypeStruct((M, T_W), jnp.int32),
        mesh=mesh,
    )
    def k(t_hbm, i_hbm, o_hbm):
        def body(i_vmem, o_vmem):
            pltpu.sync_copy(t_hbm.at[i_vmem.at[0]], o_vmem)

        pltpu.emit_pipeline(
            body,
            grid=(NW, per_w),
            in_specs=[pl.BlockSpec((1, WIN), lambda w, i: (0, w * per_w + i))],
            out_specs=[pl.BlockSpec((WIN, T_W), lambda w, i: (w * per_w + i, 0))],
            core_axis_name=("c", "s"),
            dimension_semantics=(pltpu.PARALLEL, pltpu.ARBITRARY),
        )(i_hbm, o_hbm)

    return k(table, idx_all.reshape(1, M))


def _edge_kernel(tstd, w_rbf_t, attn, w_read_t, b_read, e_real, n_blocks):
    E_pad = tstd.shape[0] // 2
    DIM = attn.shape[1]
    NG = w_rbf_t.shape[0]
    offs = [5.0 * k / (NG - 1) for k in range(NG)]
    gap = offs[1] - offs[0]
    coeff = -0.5 / (gap * gap)

    def body(ts_ref, td_ref, wr_ref, at_ref, wread_ref, bread_ref,
             sum_ref, atom_ref, be_ref):
        i = pl.program_id(0)
        ts = ts_ref[...]
        td = td_ref[...]
        H2 = DIM // 2

        def unpack_h(t):
            w = t[:, :H2]
            h1 = lax.bitcast_convert_type(
                jnp.bitwise_and(w, jnp.int32(-65536)), jnp.float32)
            h2 = lax.bitcast_convert_type(
                jnp.left_shift(w, 16), jnp.float32)
            return jnp.concatenate([h1, h2], axis=1)

        hs = unpack_h(ts) + unpack_h(td)
        e1 = jnp.where(hs > 0, hs, jnp.exp(hs) - 1.0)
        d2 = jnp.zeros((EB, 1), jnp.float32)
        for c in range(3):
            dv = lax.bitcast_convert_type(
                ts[:, H2 + c:H2 + c + 1], jnp.float32) - \
                lax.bitcast_convert_type(
                td[:, H2 + c:H2 + c + 1], jnp.float32)
            d2 = d2 + dv * dv
        dist = jnp.maximum(jnp.sqrt(d2), 0.1)
        emd = jnp.zeros((EB, DIM), jnp.float32)
        for k in range(NG):
            rk = jnp.exp(coeff * (dist - offs[k]) ** 2)
            emd = emd + rk * wr_ref[k:k + 1, :]
        e2 = e1 * emd
        z = e2 * at_ref[...]
        atom = jnp.where(z > 0, z, jnp.exp(z) - 1.0)
        logit = jnp.sum(atom * wread_ref[...], axis=1, keepdims=True)
        score = jax.nn.sigmoid(logit + bread_ref[...])
        rows = i * EB + lax.broadcasted_iota(jnp.int32, (EB, 1), 0)
        valid = rows < e_real
        y = jnp.where(valid, atom * score, 0.0)
        bi = ts[:, H2 + 3:H2 + 4]
        giota = lax.broadcasted_iota(jnp.int32, (1, G), 1)
        onehot = (bi == giota)
        part = lax.dot_general(onehot.astype(jnp.float32), y,
                               (((0,), (0,)), ((), ())),
                               preferred_element_type=jnp.float32)

        @pl.when(i == 0)
        def _():
            sum_ref[...] = jnp.zeros_like(sum_ref)

        sum_ref[...] += part
        atom_ref[...] = jnp.where(valid, atom, NEG).astype(jnp.bfloat16)
        be_ref[...] = bi.reshape(1, EB)

    return pl.pallas_call(
        body,
        grid=(n_blocks,),
        in_specs=[
            pl.BlockSpec((EB, T_W), lambda i: (i, 0)),
            pl.BlockSpec((EB, T_W), lambda i, nb=n_blocks: (nb + i, 0)),
            pl.BlockSpec((NG, DIM), lambda i: (0, 0)),
            pl.BlockSpec((1, DIM), lambda i: (0, 0)),
            pl.BlockSpec((1, DIM), lambda i: (0, 0)),
            pl.BlockSpec((1, 1), lambda i: (0, 0)),
        ],
        out_specs=[
            pl.BlockSpec((G, DIM), lambda i: (0, 0)),
            pl.BlockSpec((EB, DIM), lambda i: (i, 0)),
            pl.BlockSpec((1, EB), lambda i: (0, i)),
        ],
        out_shape=[
            jax.ShapeDtypeStruct((G, DIM), jnp.float32),
            jax.ShapeDtypeStruct((E_pad, DIM), jnp.bfloat16),
            jax.ShapeDtypeStruct((1, E_pad), jnp.int32),
        ],
        compiler_params=pltpu.CompilerParams(
            dimension_semantics=("arbitrary",)),
    )(tstd, tstd, w_rbf_t, attn, w_read_t, b_read)


def kernel(x, pos, edges_0, batch, W, b, attn, W_rbf, w_read, b_read):
    N, D_IN = x.shape
    DIM = W.shape[0]
    E = edges_0.shape[0]

    src = edges_0[:, 0]
    dst = edges_0[:, 1]
    NCHUNK = 1
    unit = WIN * NW * NCHUNK
    E_pad = ((E + unit - 1) // unit) * unit
    pad = E_pad - E
    zpad = jnp.zeros((pad,), jnp.int32)
    src_p = jnp.concatenate([src, zpad])
    dst_p = jnp.concatenate([dst, zpad])
    CE = E_pad // NCHUNK

    table = _table_kernel(x, pos, batch, W, b)

    sums, maxs = [], []
    for c in range(NCHUNK):
        sl = slice(c * CE, (c + 1) * CE)
        idx_c = jnp.concatenate([src_p[sl], dst_p[sl]])
        tstd_c = _sc_gather(table, idx_c)
        e_real_c = max(0, min(E - c * CE, CE))
        sum_c, atom_c, be_c = _edge_kernel(
            tstd_c, W_rbf.T, attn, w_read.T, b_read[None, :],
            e_real_c, CE // EB)
        sums.append(sum_c)
        maxs.append(jax.ops.segment_max(
            atom_c, be_c.reshape(-1), num_segments=G))

    out_sum = functools.reduce(jnp.add, sums)
    out_max = functools.reduce(jnp.maximum, maxs).astype(jnp.float32)
    return jnp.concatenate([out_sum, out_max], axis=1)


# speedup vs baseline: 1.4608x; 1.2477x over previous
import functools

import jax
import jax.numpy as jnp
from jax import lax
from jax.experimental import pallas as pl
from jax.experimental.pallas import tpu as pltpu
from jax.experimental.pallas import tpu_sc as plsc

T_W = 128
WIN = 128
NW = 32
EB = 2048
G = 64
NEG = -3e38


def _table_kernel(x, pos, batch, W, b):
    N, D_IN = x.shape
    DIM = W.shape[0]
    BLK = 1000
    grid = N // BLK

    def body(x_ref, pos_ref, bat_ref, w_ref, b_ref, t_ref):
        h = jnp.dot(x_ref[...], w_ref[...].T,
                    preferred_element_type=jnp.float32) + b_ref[...]
        hb = lax.bitcast_convert_type(h, jnp.int32) + 0x8000
        hi = jnp.bitwise_and(hb[:, :DIM // 2], jnp.int32(-65536))
        lo = jnp.bitwise_and(jnp.right_shift(hb[:, DIM // 2:], 16),
                             jnp.int32(0xFFFF))
        packed = jnp.bitwise_or(hi, lo)
        posb = lax.bitcast_convert_type(pos_ref[...], jnp.int32)
        z = jnp.zeros((BLK, T_W - DIM // 2 - 4), jnp.int32)
        t_ref[...] = jnp.concatenate([packed, posb, bat_ref[...], z], axis=1)

    return pl.pallas_call(
        body,
        grid=(grid,),
        in_specs=[
            pl.BlockSpec((BLK, D_IN), lambda i: (i, 0)),
            pl.BlockSpec((BLK, 3), lambda i: (i, 0)),
            pl.BlockSpec((BLK, 1), lambda i: (i, 0)),
            pl.BlockSpec((DIM, D_IN), lambda i: (0, 0)),
            pl.BlockSpec((1, DIM), lambda i: (0, 0)),
        ],
        out_specs=pl.BlockSpec((BLK, T_W), lambda i: (i, 0)),
        out_shape=jax.ShapeDtypeStruct((N, T_W), jnp.int32),
    )(x, pos, batch[:, None], W, b[None, :])


def _sc_gather(table, idx_all):
    M = idx_all.shape[0]
    mesh = plsc.VectorSubcoreMesh(core_axis_name="c", subcore_axis_name="s")
    per_w = M // WIN // NW

    @functools.partial(
        pl.kernel,
        out_type=jax.ShapeDtypeStruct((M, T_W), jnp.int32),
        mesh=mesh,
    )
    def k(t_hbm, i_hbm, o_hbm):
        def body(i_vmem, o_vmem):
            pltpu.sync_copy(t_hbm.at[i_vmem.at[0]], o_vmem)

        pltpu.emit_pipeline(
            body,
            grid=(NW, per_w),
            in_specs=[pl.BlockSpec((1, WIN), lambda w, i: (0, w * per_w + i))],
            out_specs=[pl.BlockSpec((WIN, T_W), lambda w, i: (w * per_w + i, 0))],
            core_axis_name=("c", "s"),
            dimension_semantics=(pltpu.PARALLEL, pltpu.ARBITRARY),
        )(i_hbm, o_hbm)

    return k(table, idx_all.reshape(1, M))


def _edge_kernel(tstd, w_rbf_t, attn, w_read_t, b_read, e_real, n_blocks):
    E_pad = tstd.shape[0] // 2
    DIM = attn.shape[1]
    NG = w_rbf_t.shape[0]
    offs = [5.0 * k / (NG - 1) for k in range(NG)]
    gap = offs[1] - offs[0]
    coeff = -0.5 / (gap * gap)

    def body(ts_ref, td_ref, wr_ref, at_ref, wread_ref, bread_ref,
             sum_ref, atom_ref, be_ref):
        i = pl.program_id(0)
        ts = ts_ref[...]
        td = td_ref[...]
        H2 = DIM // 2

        def unpack_h(t):
            w = t[:, :H2]
            h1 = lax.bitcast_convert_type(
                jnp.bitwise_and(w, jnp.int32(-65536)), jnp.float32)
            h2 = lax.bitcast_convert_type(
                jnp.left_shift(w, 16), jnp.float32)
            return jnp.concatenate([h1, h2], axis=1)

        hs = unpack_h(ts) + unpack_h(td)
        e1 = jnp.where(hs > 0, hs, jnp.exp(hs) - 1.0)
        d2 = jnp.zeros((EB, 1), jnp.float32)
        for c in range(3):
            dv = lax.bitcast_convert_type(
                ts[:, H2 + c:H2 + c + 1], jnp.float32) - \
                lax.bitcast_convert_type(
                td[:, H2 + c:H2 + c + 1], jnp.float32)
            d2 = d2 + dv * dv
        dist = jnp.maximum(jnp.sqrt(d2), 0.1)
        emd = jnp.zeros((EB, DIM), jnp.float32)
        for k in range(NG):
            rk = jnp.exp(coeff * (dist - offs[k]) ** 2)
            emd = emd + rk * wr_ref[k:k + 1, :]
        e2 = e1 * emd
        z = e2 * at_ref[...]
        atom = jnp.where(z > 0, z, jnp.exp(z) - 1.0)
        logit = jnp.sum(atom * wread_ref[...], axis=1, keepdims=True)
        score = jax.nn.sigmoid(logit + bread_ref[...])
        rows = i * EB + lax.broadcasted_iota(jnp.int32, (EB, 1), 0)
        valid = rows < e_real
        y = jnp.where(valid, atom * score, 0.0)
        bi = ts[:, H2 + 3:H2 + 4]
        giota = lax.broadcasted_iota(jnp.int32, (1, G), 1)
        onehot = (bi == giota)
        part = lax.dot_general(onehot.astype(jnp.float32), y,
                               (((0,), (0,)), ((), ())),
                               preferred_element_type=jnp.float32)

        @pl.when(i == 0)
        def _():
            sum_ref[...] = jnp.zeros_like(sum_ref)

        sum_ref[...] += part
        atom_ref[...] = jnp.where(valid, atom, NEG)
        be_ref[...] = bi.reshape(1, EB)

    return pl.pallas_call(
        body,
        grid=(n_blocks,),
        in_specs=[
            pl.BlockSpec((EB, T_W), lambda i: (i, 0)),
            pl.BlockSpec((EB, T_W), lambda i, nb=n_blocks: (nb + i, 0)),
            pl.BlockSpec((NG, DIM), lambda i: (0, 0)),
            pl.BlockSpec((1, DIM), lambda i: (0, 0)),
            pl.BlockSpec((1, DIM), lambda i: (0, 0)),
            pl.BlockSpec((1, 1), lambda i: (0, 0)),
        ],
        out_specs=[
            pl.BlockSpec((G, DIM), lambda i: (0, 0)),
            pl.BlockSpec((EB, DIM), lambda i: (i, 0)),
            pl.BlockSpec((1, EB), lambda i: (0, i)),
        ],
        out_shape=[
            jax.ShapeDtypeStruct((G, DIM), jnp.float32),
            jax.ShapeDtypeStruct((E_pad, DIM), jnp.float32),
            jax.ShapeDtypeStruct((1, E_pad), jnp.int32),
        ],
        compiler_params=pltpu.CompilerParams(
            dimension_semantics=("arbitrary",)),
    )(tstd, tstd, w_rbf_t, attn, w_read_t, b_read)


def _sc_segmax(atom, batch_e):
    E_pad, DIM = atom.shape
    per_w = E_pad // NW
    CH = 128
    n_ch = per_w // CH
    mesh = plsc.VectorSubcoreMesh(core_axis_name="c", subcore_axis_name="s")

    @functools.partial(
        pl.kernel,
        out_type=jax.ShapeDtypeStruct((NW, G, DIM), jnp.float32),
        mesh=mesh,
        scratch_types=[
            pltpu.VMEM((CH, DIM), jnp.float32),
            pltpu.VMEM((1, CH), jnp.int32),
            pltpu.VMEM((G, DIM), jnp.float32),
        ],
    )
    def k(a_hbm, b_hbm, o_hbm, a_v, b_v, acc):
        wid = lax.axis_index("s") * 2 + lax.axis_index("c")
        base = wid * per_w
        ninf = jnp.full((16,), -jnp.inf, jnp.float32)

        @pl.loop(0, G)
        def _(g):
            for j in range(DIM // 16):
                acc[g, pl.ds(j * 16, 16)] = ninf

        @pl.loop(0, n_ch)
        def _(ci):
            start = base + ci * CH
            pltpu.sync_copy(a_hbm.at[pl.ds(start, CH)], a_v)
            pltpu.sync_copy(b_hbm.at[:, pl.ds(start, CH)], b_v)

            @pl.loop(0, CH, step=16)
            def _(e0):
                bvec = b_v[0, pl.ds(e0, 16)]
                for j in range(16):
                    bseg = bvec[j]
                    for kf in range(DIM // 16):
                        sl = pl.ds(kf * 16, 16)
                        acc[bseg, sl] = jnp.maximum(acc[bseg, sl],
                                                    a_v[e0 + j, sl])

        pltpu.sync_copy(acc, o_hbm.at[wid])

    return k(atom, batch_e)


def kernel(x, pos, edges_0, batch, W, b, attn, W_rbf, w_read, b_read):
    N, D_IN = x.shape
    DIM = W.shape[0]
    E = edges_0.shape[0]

    src = edges_0[:, 0]
    dst = edges_0[:, 1]
    NCHUNK = 1
    unit = WIN * NW * NCHUNK
    E_pad = ((E + unit - 1) // unit) * unit
    pad = E_pad - E
    zpad = jnp.zeros((pad,), jnp.int32)
    src_p = jnp.concatenate([src, zpad])
    dst_p = jnp.concatenate([dst, zpad])
    CE = E_pad // NCHUNK

    table = _table_kernel(x, pos, batch, W, b)

    sums, maxs = [], []
    for c in range(NCHUNK):
        sl = slice(c * CE, (c + 1) * CE)
        idx_c = jnp.concatenate([src_p[sl], dst_p[sl]])
        tstd_c = _sc_gather(table, idx_c)
        e_real_c = max(0, min(E - c * CE, CE))
        sum_c, atom_c, be_c = _edge_kernel(
            tstd_c, W_rbf.T, attn, w_read.T, b_read[None, :],
            e_real_c, CE // EB)
        sums.append(sum_c)
        maxs.append(jnp.max(_sc_segmax(atom_c, be_c), axis=0))

    out_sum = functools.reduce(jnp.add, sums)
    out_max = functools.reduce(jnp.maximum, maxs)
    return jnp.concatenate([out_sum, out_max], axis=1)


# speedup vs baseline: 1.4617x; 1.0006x over previous
import functools

import jax
import jax.numpy as jnp
from jax import lax
from jax.experimental import pallas as pl
from jax.experimental.pallas import tpu as pltpu
from jax.experimental.pallas import tpu_sc as plsc

T_W = 128
WIN = 128
NW = 32
EB = 2048
G = 64
NEG = -3e38


def _table_kernel(x, pos, batch, W, b):
    N, D_IN = x.shape
    DIM = W.shape[0]
    BLK = 1000
    grid = N // BLK

    def body(x_ref, pos_ref, bat_ref, w_ref, b_ref, t_ref):
        h = jnp.dot(x_ref[...], w_ref[...].T,
                    preferred_element_type=jnp.float32) + b_ref[...]
        hb = lax.bitcast_convert_type(h, jnp.int32) + 0x8000
        hi = jnp.bitwise_and(hb[:, :DIM // 2], jnp.int32(-65536))
        lo = jnp.bitwise_and(jnp.right_shift(hb[:, DIM // 2:], 16),
                             jnp.int32(0xFFFF))
        packed = jnp.bitwise_or(hi, lo)
        posb = lax.bitcast_convert_type(pos_ref[...], jnp.int32)
        z = jnp.zeros((BLK, T_W - DIM // 2 - 4), jnp.int32)
        t_ref[...] = jnp.concatenate([packed, posb, bat_ref[...], z], axis=1)

    return pl.pallas_call(
        body,
        grid=(grid,),
        in_specs=[
            pl.BlockSpec((BLK, D_IN), lambda i: (i, 0)),
            pl.BlockSpec((BLK, 3), lambda i: (i, 0)),
            pl.BlockSpec((BLK, 1), lambda i: (i, 0)),
            pl.BlockSpec((DIM, D_IN), lambda i: (0, 0)),
            pl.BlockSpec((1, DIM), lambda i: (0, 0)),
        ],
        out_specs=pl.BlockSpec((BLK, T_W), lambda i: (i, 0)),
        out_shape=jax.ShapeDtypeStruct((N, T_W), jnp.int32),
    )(x, pos, batch[:, None], W, b[None, :])


def _sc_gather(table, idx_all):
    M = idx_all.shape[0]
    mesh = plsc.VectorSubcoreMesh(core_axis_name="c", subcore_axis_name="s")
    per_w = M // WIN // NW

    @functools.partial(
        pl.kernel,
        out_type=jax.ShapeDtypeStruct((M, T_W), jnp.int32),
        mesh=mesh,
    )
    def k(t_hbm, i_hbm, o_hbm):
        def body(i_vmem, o_vmem):
            pltpu.sync_copy(t_hbm.at[i_vmem.at[0]], o_vmem)

        pltpu.emit_pipeline(
            body,
            grid=(NW, per_w),
            in_specs=[pl.BlockSpec((1, WIN), lambda w, i: (0, w * per_w + i))],
            out_specs=[pl.BlockSpec((WIN, T_W), lambda w, i: (w * per_w + i, 0))],
            core_axis_name=("c", "s"),
            dimension_semantics=(pltpu.PARALLEL, pltpu.ARBITRARY),
        )(i_hbm, o_hbm)

    return k(table, idx_all.reshape(1, M))


def _edge_kernel(tstd, w_rbf_t, attn, w_read_t, b_read, e_real, n_blocks):
    E_pad = tstd.shape[0] // 2
    DIM = attn.shape[1]
    NG = w_rbf_t.shape[0]
    offs = [5.0 * k / (NG - 1) for k in range(NG)]
    gap = offs[1] - offs[0]
    coeff = -0.5 / (gap * gap)

    def body(ts_ref, td_ref, wr_ref, at_ref, wread_ref, bread_ref,
             sum_ref, atom_ref, be_ref):
        i = pl.program_id(0)
        ts = ts_ref[...]
        td = td_ref[...]
        H2 = DIM // 2

        def unpack_h(t):
            w = t[:, :H2]
            h1 = lax.bitcast_convert_type(
                jnp.bitwise_and(w, jnp.int32(-65536)), jnp.float32)
            h2 = lax.bitcast_convert_type(
                jnp.left_shift(w, 16), jnp.float32)
            return jnp.concatenate([h1, h2], axis=1)

        hs = unpack_h(ts) + unpack_h(td)
        e1 = jnp.where(hs > 0, hs, jnp.exp(hs) - 1.0)
        d2 = jnp.zeros((EB, 1), jnp.float32)
        for c in range(3):
            dv = lax.bitcast_convert_type(
                ts[:, H2 + c:H2 + c + 1], jnp.float32) - \
                lax.bitcast_convert_type(
                td[:, H2 + c:H2 + c + 1], jnp.float32)
            d2 = d2 + dv * dv
        dist = jnp.maximum(jnp.sqrt(d2), 0.1)
        emd = jnp.zeros((EB, DIM), jnp.float32)
        for k in range(NG):
            rk = jnp.exp(coeff * (dist - offs[k]) ** 2)
            emd = emd + rk * wr_ref[k:k + 1, :]
        e2 = e1 * emd
        z = e2 * at_ref[...]
        atom = jnp.where(z > 0, z, jnp.exp(z) - 1.0)
        logit = jnp.sum(atom * wread_ref[...], axis=1, keepdims=True)
        score = jax.nn.sigmoid(logit + bread_ref[...])
        rows = i * EB + lax.broadcasted_iota(jnp.int32, (EB, 1), 0)
        valid = rows < e_real
        y = jnp.where(valid, atom * score, 0.0)
        bi = ts[:, H2 + 3:H2 + 4]
        giota = lax.broadcasted_iota(jnp.int32, (1, G), 1)
        onehot = (bi == giota)
        part = lax.dot_general(onehot.astype(jnp.bfloat16),
                               y.astype(jnp.bfloat16),
                               (((0,), (0,)), ((), ())),
                               preferred_element_type=jnp.float32)

        @pl.when(i == 0)
        def _():
            sum_ref[...] = jnp.zeros_like(sum_ref)

        sum_ref[...] += part
        atom_ref[...] = jnp.where(valid, atom, NEG)
        be_ref[...] = bi.reshape(1, EB)

    return pl.pallas_call(
        body,
        grid=(n_blocks,),
        in_specs=[
            pl.BlockSpec((EB, T_W), lambda i: (i, 0)),
            pl.BlockSpec((EB, T_W), lambda i, nb=n_blocks: (nb + i, 0)),
            pl.BlockSpec((NG, DIM), lambda i: (0, 0)),
            pl.BlockSpec((1, DIM), lambda i: (0, 0)),
            pl.BlockSpec((1, DIM), lambda i: (0, 0)),
            pl.BlockSpec((1, 1), lambda i: (0, 0)),
        ],
        out_specs=[
            pl.BlockSpec((G, DIM), lambda i: (0, 0)),
            pl.BlockSpec((EB, DIM), lambda i: (i, 0)),
            pl.BlockSpec((1, EB), lambda i: (0, i)),
        ],
        out_shape=[
            jax.ShapeDtypeStruct((G, DIM), jnp.float32),
            jax.ShapeDtypeStruct((E_pad, DIM), jnp.float32),
            jax.ShapeDtypeStruct((1, E_pad), jnp.int32),
        ],
        compiler_params=pltpu.CompilerParams(
            dimension_semantics=("arbitrary",)),
    )(tstd, tstd, w_rbf_t, attn, w_read_t, b_read)


def _sc_segmax(atom, batch_e):
    E_pad, DIM = atom.shape
    per_w = E_pad // NW
    CH = 128
    n_ch = per_w // CH
    mesh = plsc.VectorSubcoreMesh(core_axis_name="c", subcore_axis_name="s")

    @functools.partial(
        pl.kernel,
        out_type=jax.ShapeDtypeStruct((NW, G, DIM), jnp.float32),
        mesh=mesh,
        scratch_types=[
            pltpu.VMEM((CH, DIM), jnp.float32),
            pltpu.VMEM((1, CH), jnp.int32),
            pltpu.VMEM((G, DIM), jnp.float32),
        ],
    )
    def k(a_hbm, b_hbm, o_hbm, a_v, b_v, acc):
        wid = lax.axis_index("s") * 2 + lax.axis_index("c")
        base = wid * per_w
        ninf = jnp.full((16,), -jnp.inf, jnp.float32)

        @pl.loop(0, G)
        def _(g):
            for j in range(DIM // 16):
                acc[g, pl.ds(j * 16, 16)] = ninf

        @pl.loop(0, n_ch)
        def _(ci):
            start = base + ci * CH
            pltpu.sync_copy(a_hbm.at[pl.ds(start, CH)], a_v)
            pltpu.sync_copy(b_hbm.at[:, pl.ds(start, CH)], b_v)

            @pl.loop(0, CH, step=16)
            def _(e0):
                bvec = b_v[0, pl.ds(e0, 16)]
                for j in range(16):
                    bseg = bvec[j]
                    for kf in range(DIM // 16):
                        sl = pl.ds(kf * 16, 16)
                        acc[bseg, sl] = jnp.maximum(acc[bseg, sl],
                                                    a_v[e0 + j, sl])

        pltpu.sync_copy(acc, o_hbm.at[wid])

    return k(atom, batch_e)


def kernel(x, pos, edges_0, batch, W, b, attn, W_rbf, w_read, b_read):
    N, D_IN = x.shape
    DIM = W.shape[0]
    E = edges_0.shape[0]

    src = edges_0[:, 0]
    dst = edges_0[:, 1]
    NCHUNK = 1
    unit = WIN * NW * NCHUNK
    E_pad = ((E + unit - 1) // unit) * unit
    pad = E_pad - E
    zpad = jnp.zeros((pad,), jnp.int32)
    src_p = jnp.concatenate([src, zpad])
    dst_p = jnp.concatenate([dst, zpad])
    CE = E_pad // NCHUNK

    table = _table_kernel(x, pos, batch, W, b)

    sums, maxs = [], []
    for c in range(NCHUNK):
        sl = slice(c * CE, (c + 1) * CE)
        idx_c = jnp.concatenate([src_p[sl], dst_p[sl]])
        tstd_c = _sc_gather(table, idx_c)
        e_real_c = max(0, min(E - c * CE, CE))
        sum_c, atom_c, be_c = _edge_kernel(
            tstd_c, W_rbf.T, attn, w_read.T, b_read[None, :],
            e_real_c, CE // EB)
        sums.append(sum_c)
        maxs.append(jnp.max(_sc_segmax(atom_c, be_c), axis=0))

    out_sum = functools.reduce(jnp.add, sums)
    out_max = functools.reduce(jnp.maximum, maxs)
    return jnp.concatenate([out_sum, out_max], axis=1)


# speedup vs baseline: 1.4855x; 1.0163x over previous
import functools

import jax
import jax.numpy as jnp
from jax import lax
from jax.experimental import pallas as pl
from jax.experimental.pallas import tpu as pltpu
from jax.experimental.pallas import tpu_sc as plsc

T_W = 128
WIN = 128
NW = 32
EB = 2048
G = 64
NEG = -3e38


def _table_kernel(x, pos, batch, W, b):
    N, D_IN = x.shape
    DIM = W.shape[0]
    BLK = 1000
    grid = N // BLK

    def body(x_ref, pos_ref, bat_ref, w_ref, b_ref, t_ref):
        h = jnp.dot(x_ref[...], w_ref[...].T,
                    preferred_element_type=jnp.float32) + b_ref[...]
        hb = lax.bitcast_convert_type(h, jnp.int32) + 0x8000
        hi = jnp.bitwise_and(hb[:, :DIM // 2], jnp.int32(-65536))
        lo = jnp.bitwise_and(jnp.right_shift(hb[:, DIM // 2:], 16),
                             jnp.int32(0xFFFF))
        packed = jnp.bitwise_or(hi, lo)
        posb = lax.bitcast_convert_type(pos_ref[...], jnp.int32)
        z = jnp.zeros((BLK, T_W - DIM // 2 - 4), jnp.int32)
        t_ref[...] = jnp.concatenate([packed, posb, bat_ref[...], z], axis=1)

    return pl.pallas_call(
        body,
        grid=(grid,),
        in_specs=[
            pl.BlockSpec((BLK, D_IN), lambda i: (i, 0)),
            pl.BlockSpec((BLK, 3), lambda i: (i, 0)),
            pl.BlockSpec((BLK, 1), lambda i: (i, 0)),
            pl.BlockSpec((DIM, D_IN), lambda i: (0, 0)),
            pl.BlockSpec((1, DIM), lambda i: (0, 0)),
        ],
        out_specs=pl.BlockSpec((BLK, T_W), lambda i: (i, 0)),
        out_shape=jax.ShapeDtypeStruct((N, T_W), jnp.int32),
    )(x, pos, batch[:, None], W, b[None, :])


def _sc_gather(table, idx_all):
    M = idx_all.shape[0]
    mesh = plsc.VectorSubcoreMesh(core_axis_name="c", subcore_axis_name="s")
    per_w = M // WIN // NW

    @functools.partial(
        pl.kernel,
        out_type=jax.ShapeDtypeStruct((M, T_W), jnp.int32),
        mesh=mesh,
    )
    def k(t_hbm, i_hbm, o_hbm):
        def body(i_vmem, o_vmem):
            pltpu.sync_copy(t_hbm.at[i_vmem.at[0]], o_vmem)

        pltpu.emit_pipeline(
            body,
            grid=(NW, per_w),
            in_specs=[pl.BlockSpec((1, WIN), lambda w, i: (0, w * per_w + i))],
            out_specs=[pl.BlockSpec((WIN, T_W), lambda w, i: (w * per_w + i, 0))],
            core_axis_name=("c", "s"),
            dimension_semantics=(pltpu.PARALLEL, pltpu.ARBITRARY),
        )(i_hbm, o_hbm)

    return k(table, idx_all.reshape(1, M))


def _edge_kernel(tstd, w_rbf_t, attn, w_read_t, b_read, e_real, n_blocks):
    E_pad = tstd.shape[0] // 2
    DIM = attn.shape[1]
    NG = w_rbf_t.shape[0]
    offs = [5.0 * k / (NG - 1) for k in range(NG)]
    gap = offs[1] - offs[0]
    coeff = -0.5 / (gap * gap)

    def body(ts_ref, td_ref, wr_ref, at_ref, wread_ref, bread_ref,
             sum_ref, atom_ref, be_ref):
        i = pl.program_id(0)
        ts = ts_ref[...]
        td = td_ref[...]
        H2 = DIM // 2

        def unpack_h(t):
            w = t[:, :H2]
            h1 = lax.bitcast_convert_type(
                jnp.bitwise_and(w, jnp.int32(-65536)), jnp.float32)
            h2 = lax.bitcast_convert_type(
                jnp.left_shift(w, 16), jnp.float32)
            return jnp.concatenate([h1, h2], axis=1)

        hs = unpack_h(ts) + unpack_h(td)
        e1 = jnp.where(hs > 0, hs, jnp.exp(hs) - 1.0)
        d2 = jnp.zeros((EB, 1), jnp.float32)
        for c in range(3):
            dv = lax.bitcast_convert_type(
                ts[:, H2 + c:H2 + c + 1], jnp.float32) - \
                lax.bitcast_convert_type(
                td[:, H2 + c:H2 + c + 1], jnp.float32)
            d2 = d2 + dv * dv
        dist = jnp.maximum(jnp.sqrt(d2), 0.1)
        emd = jnp.zeros((EB, DIM), jnp.float32)
        for k in range(NG):
            rk = jnp.exp(coeff * (dist - offs[k]) ** 2)
            emd = emd + rk * wr_ref[k:k + 1, :]
        e2 = e1 * emd
        z = e2 * at_ref[...]
        atom = jnp.where(z > 0, z, jnp.exp(z) - 1.0)
        logit = jnp.sum(atom * wread_ref[...], axis=1, keepdims=True)
        score = jax.nn.sigmoid(logit + bread_ref[...])
        rows = i * EB + lax.broadcasted_iota(jnp.int32, (EB, 1), 0)
        valid = rows < e_real
        y = jnp.where(valid, atom * score, 0.0)
        bi = ts[:, H2 + 3:H2 + 4]
        giota = lax.broadcasted_iota(jnp.int32, (1, G), 1)
        onehot = (bi == giota)
        part = lax.dot_general(onehot.astype(jnp.bfloat16),
                               y.astype(jnp.bfloat16),
                               (((0,), (0,)), ((), ())),
                               preferred_element_type=jnp.float32)

        @pl.when(i == 0)
        def _():
            sum_ref[...] = jnp.zeros_like(sum_ref)

        sum_ref[...] += part
        atom_ref[...] = jnp.where(valid, atom, NEG)
        be_ref[...] = bi.reshape(1, EB)

    return pl.pallas_call(
        body,
        grid=(n_blocks,),
        in_specs=[
            pl.BlockSpec((EB, T_W), lambda i: (i, 0)),
            pl.BlockSpec((EB, T_W), lambda i, nb=n_blocks: (nb + i, 0)),
            pl.BlockSpec((NG, DIM), lambda i: (0, 0)),
            pl.BlockSpec((1, DIM), lambda i: (0, 0)),
            pl.BlockSpec((1, DIM), lambda i: (0, 0)),
            pl.BlockSpec((1, 1), lambda i: (0, 0)),
        ],
        out_specs=[
            pl.BlockSpec((G, DIM), lambda i: (0, 0)),
            pl.BlockSpec((EB, DIM), lambda i: (i, 0)),
            pl.BlockSpec((1, EB), lambda i: (0, i)),
        ],
        out_shape=[
            jax.ShapeDtypeStruct((G, DIM), jnp.float32),
            jax.ShapeDtypeStruct((E_pad, DIM), jnp.float32),
            jax.ShapeDtypeStruct((1, E_pad), jnp.int32),
        ],
        compiler_params=pltpu.CompilerParams(
            dimension_semantics=("arbitrary",)),
    )(tstd, tstd, w_rbf_t, attn, w_read_t, b_read)


def _sc_segmax(atom, batch_e):
    E_pad, DIM = atom.shape
    per_w = E_pad // NW
    CH = 128
    n_ch = per_w // CH
    mesh = plsc.VectorSubcoreMesh(core_axis_name="c", subcore_axis_name="s")

    @functools.partial(
        pl.kernel,
        out_type=jax.ShapeDtypeStruct((NW, G, DIM), jnp.float32),
        mesh=mesh,
        scratch_types=[
            pltpu.VMEM((CH, DIM), jnp.float32),
            pltpu.VMEM((1, CH), jnp.int32),
            pltpu.VMEM((G, DIM), jnp.float32),
        ],
    )
    def k(a_hbm, b_hbm, o_hbm, a_v, b_v, acc):
        wid = lax.axis_index("s") * 2 + lax.axis_index("c")
        base = wid * per_w
        ninf = jnp.full((16,), -jnp.inf, jnp.float32)

        @pl.loop(0, G)
        def _(g):
            for j in range(DIM // 16):
                acc[g, pl.ds(j * 16, 16)] = ninf

        @pl.loop(0, n_ch)
        def _(ci):
            start = base + ci * CH
            pltpu.sync_copy(a_hbm.at[pl.ds(start, CH)], a_v)
            pltpu.sync_copy(b_hbm.at[:, pl.ds(start, CH)], b_v)

            @pl.loop(0, CH, step=16)
            def _(e0):
                bvec = b_v[0, pl.ds(e0, 16)]
                for j in range(16):
                    bseg = bvec[j]
                    for kf in range(DIM // 16):
                        sl = pl.ds(kf * 16, 16)
                        acc[bseg, sl] = jnp.maximum(acc[bseg, sl],
                                                    a_v[e0 + j, sl])

        pltpu.sync_copy(acc, o_hbm.at[wid])

    return k(atom, batch_e)


def kernel(x, pos, edges_0, batch, W, b, attn, W_rbf, w_read, b_read):
    N, D_IN = x.shape
    DIM = W.shape[0]
    E = edges_0.shape[0]

    src = edges_0[:, 0]
    dst = edges_0[:, 1]
    NCHUNK = 2
    unit = WIN * NW * NCHUNK
    E_pad = ((E + unit - 1) // unit) * unit
    pad = E_pad - E
    zpad = jnp.zeros((pad,), jnp.int32)
    src_p = jnp.concatenate([src, zpad])
    dst_p = jnp.concatenate([dst, zpad])
    CE = E_pad // NCHUNK

    table = _table_kernel(x, pos, batch, W, b)

    sums, maxs = [], []
    for c in range(NCHUNK):
        sl = slice(c * CE, (c + 1) * CE)
        idx_c = jnp.concatenate([src_p[sl], dst_p[sl]])
        tstd_c = _sc_gather(table, idx_c)
        e_real_c = max(0, min(E - c * CE, CE))
        sum_c, atom_c, be_c = _edge_kernel(
            tstd_c, W_rbf.T, attn, w_read.T, b_read[None, :],
            e_real_c, CE // EB)
        sums.append(sum_c)
        maxs.append(jnp.max(_sc_segmax(atom_c, be_c), axis=0))

    out_sum = functools.reduce(jnp.add, sums)
    out_max = functools.reduce(jnp.maximum, maxs)
    return jnp.concatenate([out_sum, out_max], axis=1)


# speedup vs baseline: 1.6506x; 1.1111x over previous
import functools

import jax
import jax.numpy as jnp
from jax import lax
from jax.experimental import pallas as pl
from jax.experimental.pallas import tpu as pltpu
from jax.experimental.pallas import tpu_sc as plsc

T_W = 128
WIN = 128
NW = 32
EB = 2048
G = 64
NEG = -3e38


def _table_kernel(x, pos, batch, W, b):
    N, D_IN = x.shape
    DIM = W.shape[0]
    BLK = 1000
    grid = N // BLK

    def body(x_ref, pos_ref, bat_ref, w_ref, b_ref, t_ref):
        h = jnp.dot(x_ref[...], w_ref[...].T,
                    preferred_element_type=jnp.float32) + b_ref[...]
        hb = lax.bitcast_convert_type(h, jnp.int32) + 0x8000
        hi = jnp.bitwise_and(hb[:, :DIM // 2], jnp.int32(-65536))
        lo = jnp.bitwise_and(jnp.right_shift(hb[:, DIM // 2:], 16),
                             jnp.int32(0xFFFF))
        packed = jnp.bitwise_or(hi, lo)
        posb = lax.bitcast_convert_type(pos_ref[...], jnp.int32)
        z = jnp.zeros((BLK, T_W - DIM // 2 - 4), jnp.int32)
        t_ref[...] = jnp.concatenate([packed, posb, bat_ref[...], z], axis=1)

    return pl.pallas_call(
        body,
        grid=(grid,),
        in_specs=[
            pl.BlockSpec((BLK, D_IN), lambda i: (i, 0)),
            pl.BlockSpec((BLK, 3), lambda i: (i, 0)),
            pl.BlockSpec((BLK, 1), lambda i: (i, 0)),
            pl.BlockSpec((DIM, D_IN), lambda i: (0, 0)),
            pl.BlockSpec((1, DIM), lambda i: (0, 0)),
        ],
        out_specs=pl.BlockSpec((BLK, T_W), lambda i: (i, 0)),
        out_shape=jax.ShapeDtypeStruct((N, T_W), jnp.int32),
    )(x, pos, batch[:, None], W, b[None, :])


def _sc_gather(table, idx_all):
    M = idx_all.shape[0]
    mesh = plsc.VectorSubcoreMesh(core_axis_name="c", subcore_axis_name="s")
    per_w = M // WIN // NW

    @functools.partial(
        pl.kernel,
        out_type=jax.ShapeDtypeStruct((M, T_W), jnp.int32),
        mesh=mesh,
    )
    def k(t_hbm, i_hbm, o_hbm):
        def body(i_vmem, o_vmem):
            pltpu.sync_copy(t_hbm.at[i_vmem.at[0]], o_vmem)

        pltpu.emit_pipeline(
            body,
            grid=(NW, per_w),
            in_specs=[pl.BlockSpec((1, WIN), lambda w, i: (0, w * per_w + i))],
            out_specs=[pl.BlockSpec((WIN, T_W), lambda w, i: (w * per_w + i, 0))],
            core_axis_name=("c", "s"),
            dimension_semantics=(pltpu.PARALLEL, pltpu.ARBITRARY),
        )(i_hbm, o_hbm)

    return k(table, idx_all.reshape(1, M))


def _edge_kernel(tstd, w_rbf_t, attn, w_read_t, b_read, e_real, n_blocks):
    E_pad = tstd.shape[0] // 2
    DIM = attn.shape[1]
    NG = w_rbf_t.shape[0]
    offs = [5.0 * k / (NG - 1) for k in range(NG)]
    gap = offs[1] - offs[0]
    coeff = -0.5 / (gap * gap)

    def body(ts_ref, td_ref, wr_ref, at_ref, wread_ref, bread_ref,
             sum_ref, atom_ref, be_ref):
        i = pl.program_id(0)
        ts = ts_ref[...]
        td = td_ref[...]
        H2 = DIM // 2

        def unpack_h(t):
            w = t[:, :H2]
            h1 = lax.bitcast_convert_type(
                jnp.bitwise_and(w, jnp.int32(-65536)), jnp.float32)
            h2 = lax.bitcast_convert_type(
                jnp.left_shift(w, 16), jnp.float32)
            return jnp.concatenate([h1, h2], axis=1)

        hs = unpack_h(ts) + unpack_h(td)
        e1 = jnp.where(hs > 0, hs, jnp.exp(hs) - 1.0)
        d2 = jnp.zeros((EB, 1), jnp.float32)
        for c in range(3):
            dv = lax.bitcast_convert_type(
                ts[:, H2 + c:H2 + c + 1], jnp.float32) - \
                lax.bitcast_convert_type(
                td[:, H2 + c:H2 + c + 1], jnp.float32)
            d2 = d2 + dv * dv
        dist = jnp.maximum(jnp.sqrt(d2), 0.1)
        emd = jnp.zeros((EB, DIM), jnp.float32)
        for k in range(NG):
            rk = jnp.exp(coeff * (dist - offs[k]) ** 2)
            emd = emd + rk * wr_ref[k:k + 1, :]
        e2 = e1 * emd
        z = e2 * at_ref[...]
        atom = jnp.where(z > 0, z, jnp.exp(z) - 1.0)
        logit = jnp.sum(atom * wread_ref[...], axis=1, keepdims=True)
        score = jax.nn.sigmoid(logit + bread_ref[...])
        rows = i * EB + lax.broadcasted_iota(jnp.int32, (EB, 1), 0)
        valid = rows < e_real
        y = jnp.where(valid, atom * score, 0.0)
        bi = ts[:, H2 + 3:H2 + 4]
        giota = lax.broadcasted_iota(jnp.int32, (1, G), 1)
        onehot = (bi == giota)
        part = lax.dot_general(onehot.astype(jnp.bfloat16),
                               y.astype(jnp.bfloat16),
                               (((0,), (0,)), ((), ())),
                               preferred_element_type=jnp.float32)

        @pl.when(i == 0)
        def _():
            sum_ref[...] = jnp.zeros_like(sum_ref)

        sum_ref[...] += part
        atom_ref[...] = jnp.where(valid, atom, NEG)
        be_ref[...] = bi.reshape(1, EB)

    return pl.pallas_call(
        body,
        grid=(n_blocks,),
        in_specs=[
            pl.BlockSpec((EB, T_W), lambda i: (i, 0)),
            pl.BlockSpec((EB, T_W), lambda i, nb=n_blocks: (nb + i, 0)),
            pl.BlockSpec((NG, DIM), lambda i: (0, 0)),
            pl.BlockSpec((1, DIM), lambda i: (0, 0)),
            pl.BlockSpec((1, DIM), lambda i: (0, 0)),
            pl.BlockSpec((1, 1), lambda i: (0, 0)),
        ],
        out_specs=[
            pl.BlockSpec((G, DIM), lambda i: (0, 0)),
            pl.BlockSpec((EB, DIM), lambda i: (i, 0)),
            pl.BlockSpec((1, EB), lambda i: (0, i)),
        ],
        out_shape=[
            jax.ShapeDtypeStruct((G, DIM), jnp.float32),
            jax.ShapeDtypeStruct((E_pad, DIM), jnp.float32),
            jax.ShapeDtypeStruct((1, E_pad), jnp.int32),
        ],
        compiler_params=pltpu.CompilerParams(
            dimension_semantics=("arbitrary",)),
    )(tstd, tstd, w_rbf_t, attn, w_read_t, b_read)


def _sc_segmax(atom, batch_e):
    E_pad, DIM = atom.shape
    per_w = E_pad // NW
    CH = 128
    n_ch = per_w // CH
    mesh = plsc.VectorSubcoreMesh(core_axis_name="c", subcore_axis_name="s")

    @functools.partial(
        pl.kernel,
        out_type=jax.ShapeDtypeStruct((NW, G, DIM), jnp.float32),
        mesh=mesh,
        scratch_types=[
            pltpu.VMEM((CH, DIM), jnp.float32),
            pltpu.VMEM((1, CH), jnp.int32),
            pltpu.VMEM((G, DIM), jnp.float32),
        ],
    )
    def k(a_hbm, b_hbm, o_hbm, a_v, b_v, acc):
        wid = lax.axis_index("s") * 2 + lax.axis_index("c")
        base = wid * per_w
        ninf = jnp.full((16,), -jnp.inf, jnp.float32)

        @pl.loop(0, G)
        def _(g):
            for j in range(DIM // 16):
                acc[g, pl.ds(j * 16, 16)] = ninf

        @pl.loop(0, n_ch)
        def _(ci):
            start = base + ci * CH
            pltpu.sync_copy(a_hbm.at[pl.ds(start, CH)], a_v)
            pltpu.sync_copy(b_hbm.at[:, pl.ds(start, CH)], b_v)

            @pl.loop(0, CH, step=16)
            def _(e0):
                bvec = b_v[0, pl.ds(e0, 16)]
                for j in range(16):
                    bseg = bvec[j]
                    for kf in range(DIM // 16):
                        sl = pl.ds(kf * 16, 16)
                        acc[bseg, sl] = jnp.maximum(acc[bseg, sl],
                                                    a_v[e0 + j, sl])

        pltpu.sync_copy(acc, o_hbm.at[wid])

    return k(atom, batch_e)


def kernel(x, pos, edges_0, batch, W, b, attn, W_rbf, w_read, b_read):
    N, D_IN = x.shape
    DIM = W.shape[0]
    E = edges_0.shape[0]

    src = edges_0[:, 0]
    dst = edges_0[:, 1]
    NCHUNK = 4
    unit = WIN * NW * NCHUNK
    E_pad = ((E + unit - 1) // unit) * unit
    pad = E_pad - E
    zpad = jnp.zeros((pad,), jnp.int32)
    src_p = jnp.concatenate([src, zpad])
    dst_p = jnp.concatenate([dst, zpad])
    CE = E_pad // NCHUNK

    table = _table_kernel(x, pos, batch, W, b)

    sums, maxs = [], []
    for c in range(NCHUNK):
        sl = slice(c * CE, (c + 1) * CE)
        idx_c = jnp.concatenate([src_p[sl], dst_p[sl]])
        tstd_c = _sc_gather(table, idx_c)
        e_real_c = max(0, min(E - c * CE, CE))
        sum_c, atom_c, be_c = _edge_kernel(
            tstd_c, W_rbf.T, attn, w_read.T, b_read[None, :],
            e_real_c, CE // EB)
        sums.append(sum_c)
        maxs.append(jnp.max(_sc_segmax(atom_c, be_c), axis=0))

    out_sum = functools.reduce(jnp.add, sums)
    out_max = functools.reduce(jnp.maximum, maxs)
    return jnp.concatenate([out_sum, out_max], axis=1)


# speedup vs baseline: 1.7386x; 1.0533x over previous
import functools

import jax
import jax.numpy as jnp
from jax import lax
from jax.experimental import pallas as pl
from jax.experimental.pallas import tpu as pltpu
from jax.experimental.pallas import tpu_sc as plsc

T_W = 128
WIN = 128
NW = 32
EB = 2048
G = 64
NEG = -3e38


def _table_kernel(x, pos, batch, W, b):
    N, D_IN = x.shape
    DIM = W.shape[0]
    BLK = 1000
    grid = N // BLK

    def body(x_ref, pos_ref, bat_ref, w_ref, b_ref, t_ref):
        h = jnp.dot(x_ref[...], w_ref[...].T,
                    preferred_element_type=jnp.float32) + b_ref[...]
        hb = lax.bitcast_convert_type(h, jnp.int32) + 0x8000
        hi = jnp.bitwise_and(hb[:, :DIM // 2], jnp.int32(-65536))
        lo = jnp.bitwise_and(jnp.right_shift(hb[:, DIM // 2:], 16),
                             jnp.int32(0xFFFF))
        packed = jnp.bitwise_or(hi, lo)
        posb = lax.bitcast_convert_type(pos_ref[...], jnp.int32)
        z = jnp.zeros((BLK, T_W - DIM // 2 - 4), jnp.int32)
        t_ref[...] = jnp.concatenate([packed, posb, bat_ref[...], z], axis=1)

    return pl.pallas_call(
        body,
        grid=(grid,),
        in_specs=[
            pl.BlockSpec((BLK, D_IN), lambda i: (i, 0)),
            pl.BlockSpec((BLK, 3), lambda i: (i, 0)),
            pl.BlockSpec((BLK, 1), lambda i: (i, 0)),
            pl.BlockSpec((DIM, D_IN), lambda i: (0, 0)),
            pl.BlockSpec((1, DIM), lambda i: (0, 0)),
        ],
        out_specs=pl.BlockSpec((BLK, T_W), lambda i: (i, 0)),
        out_shape=jax.ShapeDtypeStruct((N, T_W), jnp.int32),
    )(x, pos, batch[:, None], W, b[None, :])


def _sc_gather(table, idx_all):
    M = idx_all.shape[0]
    mesh = plsc.VectorSubcoreMesh(core_axis_name="c", subcore_axis_name="s")
    per_w = M // WIN // NW

    @functools.partial(
        pl.kernel,
        out_type=jax.ShapeDtypeStruct((M, T_W), jnp.int32),
        mesh=mesh,
    )
    def k(t_hbm, i_hbm, o_hbm):
        def body(i_vmem, o_vmem):
            pltpu.sync_copy(t_hbm.at[i_vmem.at[0]], o_vmem)

        pltpu.emit_pipeline(
            body,
            grid=(NW, per_w),
            in_specs=[pl.BlockSpec((1, WIN), lambda w, i: (0, w * per_w + i))],
            out_specs=[pl.BlockSpec((WIN, T_W), lambda w, i: (w * per_w + i, 0))],
            core_axis_name=("c", "s"),
            dimension_semantics=(pltpu.PARALLEL, pltpu.ARBITRARY),
        )(i_hbm, o_hbm)

    return k(table, idx_all.reshape(1, M))


def _edge_kernel(tstd, w_rbf_t, attn, w_read_t, b_read, e_real, n_blocks):
    E_pad = tstd.shape[0] // 2
    DIM = attn.shape[1]
    NG = w_rbf_t.shape[0]
    offs = [5.0 * k / (NG - 1) for k in range(NG)]
    gap = offs[1] - offs[0]
    coeff = -0.5 / (gap * gap)

    def body(ts_ref, td_ref, wr_ref, at_ref, wread_ref, bread_ref,
             sum_ref, atom_ref, be_ref):
        i = pl.program_id(0)
        ts = ts_ref[...]
        td = td_ref[...]
        H2 = DIM // 2

        def unpack_h(t):
            w = t[:, :H2]
            h1 = lax.bitcast_convert_type(
                jnp.bitwise_and(w, jnp.int32(-65536)), jnp.float32)
            h2 = lax.bitcast_convert_type(
                jnp.left_shift(w, 16), jnp.float32)
            return jnp.concatenate([h1, h2], axis=1)

        hs = unpack_h(ts) + unpack_h(td)
        e1 = jnp.where(hs > 0, hs, jnp.exp(hs) - 1.0)
        d2 = jnp.zeros((EB, 1), jnp.float32)
        for c in range(3):
            dv = lax.bitcast_convert_type(
                ts[:, H2 + c:H2 + c + 1], jnp.float32) - \
                lax.bitcast_convert_type(
                td[:, H2 + c:H2 + c + 1], jnp.float32)
            d2 = d2 + dv * dv
        dist = jnp.maximum(jnp.sqrt(d2), 0.1)
        emd = jnp.zeros((EB, DIM), jnp.float32)
        for k in range(NG):
            rk = jnp.exp(coeff * (dist - offs[k]) ** 2)
            emd = emd + rk * wr_ref[k:k + 1, :]
        e2 = e1 * emd
        z = e2 * at_ref[...]
        atom = jnp.where(z > 0, z, jnp.exp(z) - 1.0)
        logit = jnp.sum(atom * wread_ref[...], axis=1, keepdims=True)
        score = jax.nn.sigmoid(logit + bread_ref[...])
        rows = i * EB + lax.broadcasted_iota(jnp.int32, (EB, 1), 0)
        valid = rows < e_real
        y = jnp.where(valid, atom * score, 0.0)
        bi = ts[:, H2 + 3:H2 + 4]
        giota = lax.broadcasted_iota(jnp.int32, (1, G), 1)
        onehot = (bi == giota)
        part = lax.dot_general(onehot.astype(jnp.bfloat16),
                               y.astype(jnp.bfloat16),
                               (((0,), (0,)), ((), ())),
                               preferred_element_type=jnp.float32)

        @pl.when(i == 0)
        def _():
            sum_ref[...] = jnp.zeros_like(sum_ref)

        sum_ref[...] += part
        atom_ref[...] = jnp.where(valid, atom, NEG)
        be_ref[...] = bi.reshape(1, EB)

    return pl.pallas_call(
        body,
        grid=(n_blocks,),
        in_specs=[
            pl.BlockSpec((EB, T_W), lambda i: (i, 0)),
            pl.BlockSpec((EB, T_W), lambda i, nb=n_blocks: (nb + i, 0)),
            pl.BlockSpec((NG, DIM), lambda i: (0, 0)),
            pl.BlockSpec((1, DIM), lambda i: (0, 0)),
            pl.BlockSpec((1, DIM), lambda i: (0, 0)),
            pl.BlockSpec((1, 1), lambda i: (0, 0)),
        ],
        out_specs=[
            pl.BlockSpec((G, DIM), lambda i: (0, 0)),
            pl.BlockSpec((EB, DIM), lambda i: (i, 0)),
            pl.BlockSpec((1, EB), lambda i: (0, i)),
        ],
        out_shape=[
            jax.ShapeDtypeStruct((G, DIM), jnp.float32),
            jax.ShapeDtypeStruct((E_pad, DIM), jnp.float32),
            jax.ShapeDtypeStruct((1, E_pad), jnp.int32),
        ],
        compiler_params=pltpu.CompilerParams(
            dimension_semantics=("arbitrary",)),
    )(tstd, tstd, w_rbf_t, attn, w_read_t, b_read)


def _sc_segmax(atom, batch_e):
    E_pad, DIM = atom.shape
    per_w = E_pad // NW
    CH = 128
    n_ch = per_w // CH
    mesh = plsc.VectorSubcoreMesh(core_axis_name="c", subcore_axis_name="s")

    @functools.partial(
        pl.kernel,
        out_type=jax.ShapeDtypeStruct((NW, G, DIM), jnp.float32),
        mesh=mesh,
        scratch_types=[
            pltpu.VMEM((CH, DIM), jnp.float32),
            pltpu.VMEM((1, CH), jnp.int32),
            pltpu.VMEM((G, DIM), jnp.float32),
        ],
    )
    def k(a_hbm, b_hbm, o_hbm, a_v, b_v, acc):
        wid = lax.axis_index("s") * 2 + lax.axis_index("c")
        base = wid * per_w
        ninf = jnp.full((16,), -jnp.inf, jnp.float32)

        @pl.loop(0, G)
        def _(g):
            for j in range(DIM // 16):
                acc[g, pl.ds(j * 16, 16)] = ninf

        @pl.loop(0, n_ch)
        def _(ci):
            start = base + ci * CH
            pltpu.sync_copy(a_hbm.at[pl.ds(start, CH)], a_v)
            pltpu.sync_copy(b_hbm.at[:, pl.ds(start, CH)], b_v)

            @pl.loop(0, CH, step=16)
            def _(e0):
                bvec = b_v[0, pl.ds(e0, 16)]
                for j in range(16):
                    bseg = bvec[j]
                    for kf in range(DIM // 16):
                        sl = pl.ds(kf * 16, 16)
                        acc[bseg, sl] = jnp.maximum(acc[bseg, sl],
                                                    a_v[e0 + j, sl])

        pltpu.sync_copy(acc, o_hbm.at[wid])

    return k(atom, batch_e)


def kernel(x, pos, edges_0, batch, W, b, attn, W_rbf, w_read, b_read):
    N, D_IN = x.shape
    DIM = W.shape[0]
    E = edges_0.shape[0]

    src = edges_0[:, 0]
    dst = edges_0[:, 1]
    NCHUNK = 8
    unit = WIN * NW * NCHUNK
    E_pad = ((E + unit - 1) // unit) * unit
    pad = E_pad - E
    zpad = jnp.zeros((pad,), jnp.int32)
    src_p = jnp.concatenate([src, zpad])
    dst_p = jnp.concatenate([dst, zpad])
    CE = E_pad // NCHUNK

    table = _table_kernel(x, pos, batch, W, b)

    sums, maxs = [], []
    for c in range(NCHUNK):
        sl = slice(c * CE, (c + 1) * CE)
        idx_c = jnp.concatenate([src_p[sl], dst_p[sl]])
        tstd_c = _sc_gather(table, idx_c)
        e_real_c = max(0, min(E - c * CE, CE))
        sum_c, atom_c, be_c = _edge_kernel(
            tstd_c, W_rbf.T, attn, w_read.T, b_read[None, :],
            e_real_c, CE // EB)
        sums.append(sum_c)
        maxs.append(jnp.max(_sc_segmax(atom_c, be_c), axis=0))

    out_sum = functools.reduce(jnp.add, sums)
    out_max = functools.reduce(jnp.maximum, maxs)
    return jnp.concatenate([out_sum, out_max], axis=1)
